# baseline (device time: 302914 ns/iter reference)
import functools

import jax
import jax.numpy as jnp
from jax import lax
from jax.experimental import pallas as pl
from jax.experimental.pallas import tpu as pltpu


def kernel(Q, K, V):
    b, sq, h, d = Q.shape
    skv = K.shape[1]
    scale = d ** -0.5
    r = skv * h

    Qr = Q.reshape(b, h, d)
    Kr = K.reshape(b, r, d)
    Vr = V.reshape(b, r, d)

    def body(q_ref, k_ref, v_ref, o_ref, ul_send, ul_recv, send_sem, recv_sem):
        i = pl.program_id(0)
        q2 = q_ref[0]
        k2 = k_ref[0]
        v2 = v_ref[0]

        s_all = lax.dot_general(
            q2, k2,
            dimension_numbers=(((1,), (1,)), ((), ())),
            preferred_element_type=jnp.float32,
        ) * scale
        hi = lax.broadcasted_iota(jnp.int32, (h, r), 0)
        ri = lax.broadcasted_iota(jnp.int32, (h, r), 1)
        pm = jnp.where(ri % h == hi, jnp.exp(s_all), 0.0)

        u = lax.dot_general(
            pm, v2,
            dimension_numbers=(((1,), (0,)), ((), ())),
            preferred_element_type=jnp.float32,
        )
        l = jnp.sum(pm, axis=1, keepdims=True)
        ul_send[i] = jnp.concatenate(
            [u, jnp.broadcast_to(l, (h, d))], axis=1
        )

        @pl.when(i == b - 1)
        def _():
            my_x = lax.axis_index("x")
            my_y = lax.axis_index("y")
            my_z = lax.axis_index("z")
            partner = (1 - my_x, my_y, my_z)

            barrier = pltpu.get_barrier_semaphore()
            pl.semaphore_signal(
                barrier, inc=1,
                device_id=partner, device_id_type=pl.DeviceIdType.MESH,
            )
            pl.semaphore_wait(barrier, 1)

            rdma = pltpu.make_async_remote_copy(
                src_ref=ul_send, dst_ref=ul_recv,
                send_sem=send_sem, recv_sem=recv_sem,
                device_id=partner, device_id_type=pl.DeviceIdType.MESH,
            )
            rdma.start()
            rdma.wait()

            tot = ul_send[...] + ul_recv[...]
            o_ref[...] = tot[:, :, :d] / tot[:, :, d:]

    O = pl.pallas_call(
        body,
        grid=(b,),
        in_specs=[
            pl.BlockSpec((1, h, d), lambda i: (i, 0, 0)),
            pl.BlockSpec((1, r, d), lambda i: (i, 0, 0)),
            pl.BlockSpec((1, r, d), lambda i: (i, 0, 0)),
        ],
        out_specs=pl.BlockSpec((b, h, d), lambda i: (0, 0, 0)),
        out_shape=jax.ShapeDtypeStruct((b, h, d), jnp.float32),
        scratch_shapes=[
            pltpu.VMEM((b, h, 2 * d), jnp.float32),
            pltpu.VMEM((b, h, 2 * d), jnp.float32),
            pltpu.SemaphoreType.DMA,
            pltpu.SemaphoreType.DMA,
        ],
        compiler_params=pltpu.CompilerParams(
            collective_id=0,
            dimension_semantics=("arbitrary",),
            vmem_limit_bytes=100 * 1024 * 1024,
        ),
    )(Qr, Kr, Vr)
    return O.reshape(b, sq, h, d)
